# Initial kernel scaffold; baseline (speedup 1.0000x reference)
import jax
import jax.numpy as jnp
from jax.experimental import pallas as pl


def kernel(x, edge_index, W1, b1, W2, b2):
    raise NotImplementedError("write your pallas kernel here")


# speedup vs baseline: 29.7652x; 29.7652x over previous
import functools

import jax
import jax.numpy as jnp
from jax import lax
from jax.experimental import pallas as pl
from jax.experimental.pallas import tpu as pltpu
from jax.experimental.pallas import tpu_sc as plsc

NC = 2
NS = 16
NW = NC * NS
CHUNK = 128


def _mesh():
    return plsc.VectorSubcoreMesh(core_axis_name="c", subcore_axis_name="s")


_SC_PARAMS = pltpu.CompilerParams(use_tc_tiling_on_sc=False)


def _make_deg_kernel(n_pad: int, chunks_total: int):
    per_w = chunks_total // NW
    rows_per_tile = n_pad // NS

    @functools.partial(
        pl.kernel,
        out_type=jax.ShapeDtypeStruct((NC, NS, rows_per_tile), jnp.float32),
        mesh=_mesh(),
        scratch_types=[
            pltpu.VMEM_SHARED((n_pad,), jnp.float32),
            pltpu.VMEM((per_w, CHUNK), jnp.int32),
            pltpu.VMEM((CHUNK,), jnp.float32),
        ],
        compiler_params=_SC_PARAMS,
    )
    def deg_kernel(dst_hbm, zeros_hbm, out_hbm, deg_sp, idx_v, ones_v):
        cid = lax.axis_index("c")
        sid = lax.axis_index("s")
        wid = sid * NC + cid
        pltpu.sync_copy(zeros_hbm.at[pl.ds(sid * rows_per_tile, rows_per_tile)],
                        deg_sp.at[pl.ds(sid * rows_per_tile, rows_per_tile)])
        pltpu.sync_copy(dst_hbm.at[pl.ds(wid * per_w, per_w)], idx_v)
        for k in range(CHUNK // 16):
            ones_v[pl.ds(k * 16, 16)] = jnp.ones((16,), jnp.float32)
        plsc.subcore_barrier()

        def body(j, carry):
            pltpu.sync_copy(ones_v, deg_sp.at[idx_v.at[j]], add=True)
            return carry

        lax.fori_loop(0, per_w, body, 0)
        plsc.subcore_barrier()
        pltpu.sync_copy(deg_sp.at[pl.ds(sid * rows_per_tile, rows_per_tile)],
                        out_hbm.at[cid, sid])

    return deg_kernel


def _make_agg_kernel(n_pad: int, f: int, chunks_total: int):
    per_w = chunks_total // NW
    rows_per_tile = n_pad // NS

    @functools.partial(
        pl.kernel,
        out_type=jax.ShapeDtypeStruct((NC, NS, rows_per_tile, f), jnp.float32),
        mesh=_mesh(),
        scratch_types=[
            pltpu.VMEM_SHARED((n_pad, f), jnp.float32),
            pltpu.VMEM((per_w, CHUNK), jnp.int32),
            pltpu.VMEM((per_w, CHUNK), jnp.int32),
            pltpu.VMEM((2, CHUNK, f), jnp.float32),
            pltpu.SemaphoreType.DMA,
            pltpu.SemaphoreType.DMA,
        ],
        compiler_params=_SC_PARAMS,
    )
    def agg_kernel(g_hbm, src_hbm, dst_hbm, zeros_hbm, out_hbm,
                   agg_sp, src_v, dst_v, rows_v, gsem, ssem):
        cid = lax.axis_index("c")
        sid = lax.axis_index("s")
        wid = sid * NC + cid
        pltpu.sync_copy(zeros_hbm.at[pl.ds(sid * rows_per_tile, rows_per_tile)],
                        agg_sp.at[pl.ds(sid * rows_per_tile, rows_per_tile)])
        pltpu.sync_copy(src_hbm.at[pl.ds(wid * per_w, per_w)], src_v)
        pltpu.sync_copy(dst_hbm.at[pl.ds(wid * per_w, per_w)], dst_v)
        plsc.subcore_barrier()

        pltpu.async_copy(g_hbm.at[src_v.at[0]], rows_v.at[0], gsem)

        def body(jj, carry):
            j = jj * 2

            def step(j, buf, nbuf):
                pltpu.make_async_copy(
                    g_hbm.at[src_v.at[j]], rows_v.at[buf], gsem).wait()

                @pl.when(j + 1 < per_w)
                def _():
                    pltpu.async_copy(
                        g_hbm.at[src_v.at[j + 1]], rows_v.at[nbuf], gsem)

                pltpu.async_copy(
                    rows_v.at[buf], agg_sp.at[dst_v.at[j]], ssem, add=True)
                pltpu.make_async_copy(
                    rows_v.at[buf], agg_sp.at[dst_v.at[j]], ssem).wait()

            step(j, 0, 1)
            step(j + 1, 1, 0)
            return carry

        lax.fori_loop(0, per_w // 2, body, 0)
        plsc.subcore_barrier()
        pltpu.sync_copy(agg_sp.at[pl.ds(sid * rows_per_tile, rows_per_tile)],
                        out_hbm.at[cid, sid])

    return agg_kernel


def _tc1_body(dp_ref, x_ref, w1_ref, dinv_ref, g1_ref):
    deg = dp_ref[0] + dp_ref[1] + 1.0
    dinv = lax.rsqrt(deg)
    dinv_ref[...] = dinv
    h = jnp.dot(x_ref[...], w1_ref[...], preferred_element_type=jnp.float32)
    g1_ref[...] = h * dinv


def _tc2_body(n_valid, agg_ref, g1_ref, dinv_ref, b1_ref, w2_ref, g2_ref):
    dinv = dinv_ref[...]
    s = agg_ref[0] + agg_ref[1] + g1_ref[...]
    z = jnp.maximum(dinv * s + b1_ref[...], 0.0)
    row = lax.broadcasted_iota(jnp.int32, dinv.shape, 0)
    z = jnp.where(row < n_valid, z, 0.0)
    g2_ref[...] = jnp.dot(z, w2_ref[...],
                          preferred_element_type=jnp.float32) * dinv


def _tc3_body(agg_ref, g2_ref, dinv_ref, b2_ref, out_ref):
    y = dinv_ref[...] * (agg_ref[0] + agg_ref[1] + g2_ref[...]) + b2_ref[...]
    m = jnp.max(y, axis=1, keepdims=True)
    lse = m + jnp.log(jnp.sum(jnp.exp(y - m), axis=1, keepdims=True))
    out_ref[...] = y - lse


def kernel(x, edge_index, W1, b1, W2, b2):
    n, d = x.shape
    e = edge_index.shape[1]
    h = W1.shape[1]
    c = W2.shape[1]

    n_pad = ((n + 511) // 512 + (1 if n % 512 == 0 else 0)) * 512
    e_pad = ((e + NW * CHUNK * 8 - 1) // (NW * CHUNK * 8)) * (NW * CHUNK * 8)
    chunks_total = e_pad // CHUNK

    src = jnp.full((e_pad,), n, dtype=jnp.int32).at[:e].set(edge_index[0])
    dst = jnp.full((e_pad,), n, dtype=jnp.int32).at[:e].set(edge_index[1])
    src2d = src.reshape(chunks_total, CHUNK)
    dst2d = dst.reshape(chunks_total, CHUNK)
    x_pad = jnp.zeros((n_pad, d), x.dtype).at[:n].set(x)
    zeros_deg = jnp.zeros((n_pad,), jnp.float32)
    zeros_rows = jnp.zeros((n_pad, h), jnp.float32)
    b1r = b1.reshape(1, h)
    b2r = b2.reshape(1, c)

    deg_parts = _make_deg_kernel(n_pad, chunks_total)(dst2d, zeros_deg)
    deg_parts = deg_parts.reshape(NC, n_pad, 1)

    dinv, g1 = pl.pallas_call(
        _tc1_body,
        out_shape=(jax.ShapeDtypeStruct((n_pad, 1), jnp.float32),
                   jax.ShapeDtypeStruct((n_pad, h), jnp.float32)),
    )(deg_parts, x_pad, W1)

    agg_fn = _make_agg_kernel(n_pad, h, chunks_total)

    agg1 = agg_fn(g1, src2d, dst2d, zeros_rows).reshape(NC, n_pad, h)

    g2 = pl.pallas_call(
        functools.partial(_tc2_body, n),
        out_shape=jax.ShapeDtypeStruct((n_pad, c), jnp.float32),
    )(agg1, g1, dinv, b1r, W2)

    agg2 = agg_fn(g2, src2d, dst2d, zeros_rows).reshape(NC, n_pad, c)

    out = pl.pallas_call(
        _tc3_body,
        out_shape=jax.ShapeDtypeStruct((n_pad, c), jnp.float32),
    )(agg2, g2, dinv, b2r)

    return out[:n]


# speedup vs baseline: 56.0792x; 1.8841x over previous
import functools

import jax
import jax.numpy as jnp
from jax import lax
from jax.experimental import pallas as pl
from jax.experimental.pallas import tpu as pltpu
from jax.experimental.pallas import tpu_sc as plsc

NC = 2
NS = 16
NW = NC * NS
CHUNK = 128


def _mesh():
    return plsc.VectorSubcoreMesh(core_axis_name="c", subcore_axis_name="s")


_SC_PARAMS = pltpu.CompilerParams(use_tc_tiling_on_sc=False)


def _make_deg_kernel(n_pad: int, chunks_total: int):
    per_w = chunks_total // NW
    rows_per_tile = n_pad // NS

    @functools.partial(
        pl.kernel,
        out_type=jax.ShapeDtypeStruct((NC, NS, rows_per_tile), jnp.float32),
        mesh=_mesh(),
        scratch_types=[
            pltpu.VMEM_SHARED((n_pad,), jnp.float32),
            pltpu.VMEM((per_w, CHUNK), jnp.int32),
            pltpu.VMEM((CHUNK,), jnp.float32),
        ],
        compiler_params=_SC_PARAMS,
    )
    def deg_kernel(dst_hbm, zeros_hbm, out_hbm, deg_sp, idx_v, ones_v):
        cid = lax.axis_index("c")
        sid = lax.axis_index("s")
        wid = sid * NC + cid
        pltpu.sync_copy(zeros_hbm.at[pl.ds(sid * rows_per_tile, rows_per_tile)],
                        deg_sp.at[pl.ds(sid * rows_per_tile, rows_per_tile)])
        pltpu.sync_copy(dst_hbm.at[pl.ds(wid * per_w, per_w)], idx_v)
        for k in range(CHUNK // 16):
            ones_v[pl.ds(k * 16, 16)] = jnp.ones((16,), jnp.float32)
        plsc.subcore_barrier()

        def body(j, carry):
            pltpu.sync_copy(ones_v, deg_sp.at[idx_v.at[j]], add=True)
            return carry

        lax.fori_loop(0, per_w, body, 0)
        plsc.subcore_barrier()
        pltpu.sync_copy(deg_sp.at[pl.ds(sid * rows_per_tile, rows_per_tile)],
                        out_hbm.at[cid, sid])

    return deg_kernel


def _make_agg_kernel(n_pad: int, f: int, chunks_total: int):
    per_w = chunks_total // NW
    rows_per_tile = n_pad // NS

    nbuf = 4

    @functools.partial(
        pl.kernel,
        out_type=jax.ShapeDtypeStruct((NC, NS, rows_per_tile, f), jnp.float32),
        mesh=_mesh(),
        scratch_types=[
            pltpu.VMEM_SHARED((n_pad, f), jnp.float32),
            pltpu.VMEM_SHARED((n_pad, f), jnp.float32),
            pltpu.VMEM((per_w, CHUNK), jnp.int32),
            pltpu.VMEM((per_w, CHUNK), jnp.int32),
            pltpu.VMEM((nbuf, CHUNK, f), jnp.float32),
            pltpu.SemaphoreType.DMA,
            pltpu.SemaphoreType.DMA,
        ],
        compiler_params=_SC_PARAMS,
    )
    def agg_kernel(g_hbm, src_hbm, dst_hbm, zeros_hbm, out_hbm,
                   agg_sp, g_sp, src_v, dst_v, rows_v, gsem, ssem):
        cid = lax.axis_index("c")
        sid = lax.axis_index("s")
        wid = sid * NC + cid
        sl = pl.ds(sid * rows_per_tile, rows_per_tile)
        pltpu.sync_copy(zeros_hbm.at[sl], agg_sp.at[sl])
        pltpu.sync_copy(g_hbm.at[sl], g_sp.at[sl])
        pltpu.sync_copy(src_hbm.at[pl.ds(wid * per_w, per_w)], src_v)
        pltpu.sync_copy(dst_hbm.at[pl.ds(wid * per_w, per_w)], dst_v)
        plsc.subcore_barrier()

        for k in range(nbuf - 1):
            pltpu.async_copy(g_sp.at[src_v.at[k]], rows_v.at[k], gsem)

        def step(j, b):
            pltpu.make_async_copy(
                g_sp.at[src_v.at[j]], rows_v.at[b], gsem).wait()
            pltpu.async_copy(
                rows_v.at[b], agg_sp.at[dst_v.at[j]], ssem, add=True)

            @pl.when(j >= 1)
            def _():
                pb = (b + nbuf - 1) % nbuf
                pltpu.make_async_copy(
                    rows_v.at[pb], agg_sp.at[dst_v.at[j - 1]], ssem).wait()

            @pl.when(j + nbuf - 1 < per_w)
            def _():
                nb = (b + nbuf - 1) % nbuf
                pltpu.async_copy(
                    g_sp.at[src_v.at[j + nbuf - 1]], rows_v.at[nb], gsem)

        def body(jj, carry):
            j = jj * nbuf
            for k in range(nbuf):
                step(j + k, k)
            return carry

        lax.fori_loop(0, per_w // nbuf, body, 0)
        pltpu.make_async_copy(
            rows_v.at[(per_w - 1) % nbuf],
            agg_sp.at[dst_v.at[per_w - 1]], ssem).wait()
        plsc.subcore_barrier()
        pltpu.sync_copy(agg_sp.at[sl], out_hbm.at[cid, sid])

    return agg_kernel


def _tc1_body(dp_ref, x_ref, w1_ref, dinv_ref, g1_ref):
    deg = dp_ref[0] + dp_ref[1] + 1.0
    dinv = lax.rsqrt(deg)
    dinv_ref[...] = dinv
    h = jnp.dot(x_ref[...], w1_ref[...], preferred_element_type=jnp.float32)
    g1_ref[...] = h * dinv


def _tc2_body(n_valid, agg_ref, g1_ref, dinv_ref, b1_ref, w2_ref, g2_ref):
    dinv = dinv_ref[...]
    s = agg_ref[0] + agg_ref[1] + g1_ref[...]
    z = jnp.maximum(dinv * s + b1_ref[...], 0.0)
    row = lax.broadcasted_iota(jnp.int32, dinv.shape, 0)
    z = jnp.where(row < n_valid, z, 0.0)
    g2_ref[...] = jnp.dot(z, w2_ref[...],
                          preferred_element_type=jnp.float32) * dinv


def _tc3_body(agg_ref, g2_ref, dinv_ref, b2_ref, out_ref):
    y = dinv_ref[...] * (agg_ref[0] + agg_ref[1] + g2_ref[...]) + b2_ref[...]
    m = jnp.max(y, axis=1, keepdims=True)
    lse = m + jnp.log(jnp.sum(jnp.exp(y - m), axis=1, keepdims=True))
    out_ref[...] = y - lse


def kernel(x, edge_index, W1, b1, W2, b2):
    n, d = x.shape
    e = edge_index.shape[1]
    h = W1.shape[1]
    c = W2.shape[1]

    n_pad = ((n + 511) // 512 + (1 if n % 512 == 0 else 0)) * 512
    e_pad = ((e + NW * CHUNK * 8 - 1) // (NW * CHUNK * 8)) * (NW * CHUNK * 8)
    chunks_total = e_pad // CHUNK

    pad_idx = (n + jnp.arange(e_pad, dtype=jnp.int32) % (n_pad - n))
    src = pad_idx.at[:e].set(edge_index[0])
    dst = pad_idx.at[:e].set(edge_index[1])
    src2d = src.reshape(chunks_total, CHUNK)
    dst2d = dst.reshape(chunks_total, CHUNK)
    x_pad = jnp.zeros((n_pad, d), x.dtype).at[:n].set(x)
    zeros_deg = jnp.zeros((n_pad,), jnp.float32)
    zeros_rows = jnp.zeros((n_pad, h), jnp.float32)
    b1r = b1.reshape(1, h)
    b2r = b2.reshape(1, c)

    deg_parts = _make_deg_kernel(n_pad, chunks_total)(dst2d, zeros_deg)
    deg_parts = deg_parts.reshape(NC, n_pad, 1)

    dinv, g1 = pl.pallas_call(
        _tc1_body,
        out_shape=(jax.ShapeDtypeStruct((n_pad, 1), jnp.float32),
                   jax.ShapeDtypeStruct((n_pad, h), jnp.float32)),
    )(deg_parts, x_pad, W1)

    agg_fn = _make_agg_kernel(n_pad, h, chunks_total)

    agg1 = agg_fn(g1, src2d, dst2d, zeros_rows).reshape(NC, n_pad, h)

    g2 = pl.pallas_call(
        functools.partial(_tc2_body, n),
        out_shape=jax.ShapeDtypeStruct((n_pad, c), jnp.float32),
    )(agg1, g1, dinv, b1r, W2)

    agg2 = agg_fn(g2, src2d, dst2d, zeros_rows).reshape(NC, n_pad, c)

    out = pl.pallas_call(
        _tc3_body,
        out_shape=jax.ShapeDtypeStruct((n_pad, c), jnp.float32),
    )(agg2, g2, dinv, b2r)

    return out[:n]


# speedup vs baseline: 58.8864x; 1.0501x over previous
import functools

import jax
import jax.numpy as jnp
from jax import lax
from jax.experimental import pallas as pl
from jax.experimental.pallas import tpu as pltpu
from jax.experimental.pallas import tpu_sc as plsc

NC = 2
NS = 16
NW = NC * NS
CHUNK = 128
F = 16


def _mesh():
    return plsc.VectorSubcoreMesh(core_axis_name="c", subcore_axis_name="s")


_SC_PARAMS = pltpu.CompilerParams(use_tc_tiling_on_sc=False)


def _make_deg_kernel(n_pad: int, chunks_total: int):
    per_w = chunks_total // NW
    rows_per_tile = n_pad // NS
    depth = 8

    @functools.partial(
        pl.kernel,
        out_type=jax.ShapeDtypeStruct((NC, n_pad, F), jnp.float32),
        mesh=_mesh(),
        scratch_types=[
            pltpu.VMEM_SHARED((n_pad, F), jnp.float32),
            pltpu.VMEM((per_w, CHUNK), jnp.int32),
            pltpu.VMEM((CHUNK, F), jnp.float32),
            pltpu.SemaphoreType.DMA,
        ],
        compiler_params=_SC_PARAMS,
    )
    def deg_kernel(dst_hbm, zeros_hbm, ones_hbm, out_hbm,
                   deg_sp, idx_v, ones_v, ssem):
        cid = lax.axis_index("c")
        sid = lax.axis_index("s")
        wid = sid * NC + cid
        sl = pl.ds(sid * rows_per_tile, rows_per_tile)
        pltpu.sync_copy(zeros_hbm.at[sl], deg_sp.at[sl])
        pltpu.sync_copy(dst_hbm.at[pl.ds(wid * per_w, per_w)], idx_v)
        pltpu.sync_copy(ones_hbm, ones_v)
        plsc.subcore_barrier()

        def body(j, carry):
            pltpu.async_copy(ones_v, deg_sp.at[idx_v.at[j]], ssem, add=True)

            @pl.when(j >= depth)
            def _():
                pltpu.make_async_copy(
                    ones_v, deg_sp.at[idx_v.at[j - depth]], ssem).wait()

            return carry

        lax.fori_loop(0, per_w, body, 0)
        for k in range(depth):
            pltpu.make_async_copy(
                ones_v, deg_sp.at[idx_v.at[per_w - depth + k]], ssem).wait()
        plsc.subcore_barrier()
        pltpu.sync_copy(deg_sp.at[sl], out_hbm.at[cid, sl])

    return deg_kernel


def _make_agg_kernel(n_pad: int, chunks_total: int):
    per_w = chunks_total // NW
    rows_per_tile = n_pad // NS
    nbuf = 4

    @functools.partial(
        pl.kernel,
        out_type=jax.ShapeDtypeStruct((NC, n_pad, F), jnp.float32),
        mesh=_mesh(),
        scratch_types=[
            pltpu.VMEM_SHARED((n_pad, F), jnp.float32),
            pltpu.VMEM_SHARED((n_pad, F), jnp.float32),
            pltpu.VMEM((per_w, CHUNK), jnp.int32),
            pltpu.VMEM((per_w, CHUNK), jnp.int32),
            pltpu.VMEM((nbuf, CHUNK, F), jnp.float32),
            pltpu.SemaphoreType.DMA,
            pltpu.SemaphoreType.DMA,
        ],
        compiler_params=_SC_PARAMS,
    )
    def agg_kernel(g_hbm, src_hbm, dst_hbm, zeros_hbm, out_hbm,
                   agg_sp, g_sp, src_v, dst_v, rows_v, gsem, ssem):
        cid = lax.axis_index("c")
        sid = lax.axis_index("s")
        wid = sid * NC + cid
        sl = pl.ds(sid * rows_per_tile, rows_per_tile)
        pltpu.sync_copy(zeros_hbm.at[sl], agg_sp.at[sl])
        pltpu.sync_copy(g_hbm.at[sl], g_sp.at[sl])
        pltpu.sync_copy(src_hbm.at[pl.ds(wid * per_w, per_w)], src_v)
        pltpu.sync_copy(dst_hbm.at[pl.ds(wid * per_w, per_w)], dst_v)
        plsc.subcore_barrier()

        for k in range(nbuf - 1):
            pltpu.async_copy(g_sp.at[src_v.at[k]], rows_v.at[k], gsem)

        def step(j, b):
            pltpu.make_async_copy(
                g_sp.at[src_v.at[j]], rows_v.at[b], gsem).wait()
            pltpu.async_copy(
                rows_v.at[b], agg_sp.at[dst_v.at[j]], ssem, add=True)

            @pl.when(j >= 1)
            def _():
                pb = (b + nbuf - 1) % nbuf
                pltpu.make_async_copy(
                    rows_v.at[pb], agg_sp.at[dst_v.at[j - 1]], ssem).wait()

            @pl.when(j + nbuf - 1 < per_w)
            def _():
                nb = (b + nbuf - 1) % nbuf
                pltpu.async_copy(
                    g_sp.at[src_v.at[j + nbuf - 1]], rows_v.at[nb], gsem)

        def body(jj, carry):
            j = jj * nbuf
            for k in range(nbuf):
                step(j + k, k)
            return carry

        lax.fori_loop(0, per_w // nbuf, body, 0)
        pltpu.make_async_copy(
            rows_v.at[(per_w - 1) % nbuf],
            agg_sp.at[dst_v.at[per_w - 1]], ssem).wait()
        plsc.subcore_barrier()
        pltpu.sync_copy(agg_sp.at[sl], out_hbm.at[cid, sl])

    return agg_kernel


def _tc1_body(dp_ref, x_ref, w1_ref, dinv_ref, g1_ref):
    deg = dp_ref[0] + dp_ref[1] + 1.0
    dinv = lax.rsqrt(deg)
    dinv_ref[...] = dinv
    h = jnp.dot(x_ref[...], w1_ref[...], preferred_element_type=jnp.float32)
    g1_ref[...] = h * dinv


def _tc2_body(n_valid, agg_ref, g1_ref, dinv_ref, b1_ref, w2_ref, g2_ref):
    dinv = dinv_ref[...]
    s = agg_ref[0] + agg_ref[1] + g1_ref[...]
    z = jnp.maximum(dinv * s + b1_ref[...], 0.0)
    row = lax.broadcasted_iota(jnp.int32, z.shape, 0)
    z = jnp.where(row < n_valid, z, 0.0)
    g2_ref[...] = jnp.dot(z, w2_ref[...],
                          preferred_element_type=jnp.float32) * dinv


def _tc3_body(n_valid, agg_ref, g2_ref, dinv_ref, b2_ref, out_ref):
    y = dinv_ref[...] * (agg_ref[0] + agg_ref[1] + g2_ref[...]) + b2_ref[...]
    m = jnp.max(y, axis=1, keepdims=True)
    lse = m + jnp.log(jnp.sum(jnp.exp(y - m), axis=1, keepdims=True))
    out_ref[...] = (y - lse)[:n_valid]


def kernel(x, edge_index, W1, b1, W2, b2):
    n, d = x.shape
    e = edge_index.shape[1]
    h = W1.shape[1]
    c = W2.shape[1]
    assert h == F and c == F

    n_pad = ((n + 511) // 512 + (1 if n % 512 == 0 else 0)) * 512
    e_pad = ((e + NW * CHUNK * 8 - 1) // (NW * CHUNK * 8)) * (NW * CHUNK * 8)
    chunks_total = e_pad // CHUNK

    n_spread = min(128, n_pad - n)
    pad_idx = (n + (jnp.arange(e_pad, dtype=jnp.int32) & (n_spread - 1)))
    src = pad_idx.at[:e].set(edge_index[0])
    dst = pad_idx.at[:e].set(edge_index[1])
    src2d = src.reshape(chunks_total, CHUNK)
    dst2d = dst.reshape(chunks_total, CHUNK)
    x_pad = jnp.zeros((n_pad, d), x.dtype).at[:n].set(x)
    zeros_rows = jnp.zeros((n_pad, F), jnp.float32)
    ones_rows = jnp.ones((CHUNK, F), jnp.float32)
    b1r = b1.reshape(1, F)
    b2r = b2.reshape(1, F)

    deg_parts = _make_deg_kernel(n_pad, chunks_total)(
        dst2d, zeros_rows, ones_rows)

    dinv, g1 = pl.pallas_call(
        _tc1_body,
        out_shape=(jax.ShapeDtypeStruct((n_pad, F), jnp.float32),
                   jax.ShapeDtypeStruct((n_pad, F), jnp.float32)),
    )(deg_parts, x_pad, W1)

    agg_fn = _make_agg_kernel(n_pad, chunks_total)

    agg1 = agg_fn(g1, src2d, dst2d, zeros_rows)

    g2 = pl.pallas_call(
        functools.partial(_tc2_body, n),
        out_shape=jax.ShapeDtypeStruct((n_pad, F), jnp.float32),
    )(agg1, g1, dinv, b1r, W2)

    agg2 = agg_fn(g2, src2d, dst2d, zeros_rows)

    out = pl.pallas_call(
        functools.partial(_tc3_body, n),
        out_shape=jax.ShapeDtypeStruct((n, F), jnp.float32),
    )(agg2, g2, dinv, b2r)

    return out


# speedup vs baseline: 89.4317x; 1.5187x over previous
import functools

import jax
import jax.numpy as jnp
from jax import lax
from jax.experimental import pallas as pl
from jax.experimental.pallas import tpu as pltpu
from jax.experimental.pallas import tpu_sc as plsc

NC = 2
NS = 16
NW = NC * NS
CHUNK = 128
F = 16
PACK = 128 // F


def _mesh():
    return plsc.VectorSubcoreMesh(core_axis_name="c", subcore_axis_name="s")


_SC_PARAMS = pltpu.CompilerParams(use_tc_tiling_on_sc=False)


def _make_deg_kernel(n_pad: int, chunks_total: int):
    per_w = chunks_total // NW
    rows_per_tile = n_pad // NS
    depth = 8

    @functools.partial(
        pl.kernel,
        out_type=jax.ShapeDtypeStruct((NC, n_pad, F), jnp.float32),
        mesh=_mesh(),
        scratch_types=[
            pltpu.VMEM_SHARED((n_pad, F), jnp.float32),
            pltpu.VMEM((per_w, CHUNK), jnp.int32),
            pltpu.VMEM((CHUNK, F), jnp.float32),
            pltpu.SemaphoreType.DMA,
        ],
        compiler_params=_SC_PARAMS,
    )
    def deg_kernel(ei_hbm, zeros_hbm, ones_hbm, out_hbm,
                   deg_sp, idx_v, ones_v, ssem):
        cid = lax.axis_index("c")
        sid = lax.axis_index("s")
        wid = sid * NC + cid
        sl = pl.ds(sid * rows_per_tile, rows_per_tile)
        pltpu.sync_copy(zeros_hbm.at[sl], deg_sp.at[sl])
        pltpu.sync_copy(ei_hbm.at[1, pl.ds(wid * per_w, per_w)], idx_v)
        pltpu.sync_copy(ones_hbm, ones_v)
        plsc.subcore_barrier()

        def body(j, carry):
            pltpu.async_copy(ones_v, deg_sp.at[idx_v.at[j]], ssem, add=True)

            @pl.when(j >= depth)
            def _():
                pltpu.make_async_copy(
                    ones_v, deg_sp.at[idx_v.at[j - depth]], ssem).wait()

            return carry

        lax.fori_loop(0, per_w, body, 0)
        for k in range(depth):
            pltpu.make_async_copy(
                ones_v, deg_sp.at[idx_v.at[per_w - depth + k]], ssem).wait()
        plsc.subcore_barrier()
        pltpu.sync_copy(deg_sp.at[sl], out_hbm.at[cid, sl])

    return deg_kernel


def _make_agg_kernel(n_pad: int, chunks_total: int):
    per_w = chunks_total // NW
    rows_per_tile = n_pad // NS
    nbuf = 4

    @functools.partial(
        pl.kernel,
        out_type=jax.ShapeDtypeStruct((NC, n_pad, F), jnp.float32),
        mesh=_mesh(),
        scratch_types=[
            pltpu.VMEM_SHARED((n_pad, F), jnp.float32),
            pltpu.VMEM_SHARED((n_pad, F), jnp.float32),
            pltpu.VMEM((per_w, CHUNK), jnp.int32),
            pltpu.VMEM((per_w, CHUNK), jnp.int32),
            pltpu.VMEM((nbuf, CHUNK, F), jnp.float32),
            pltpu.SemaphoreType.DMA,
            pltpu.SemaphoreType.DMA,
        ],
        compiler_params=_SC_PARAMS,
    )
    def agg_kernel(g_hbm, ei_hbm, zeros_hbm, out_hbm,
                   agg_sp, g_sp, src_v, dst_v, rows_v, gsem, ssem):
        cid = lax.axis_index("c")
        sid = lax.axis_index("s")
        wid = sid * NC + cid
        sl = pl.ds(sid * rows_per_tile, rows_per_tile)
        pltpu.sync_copy(zeros_hbm.at[sl], agg_sp.at[sl])
        pltpu.sync_copy(g_hbm.at[sl], g_sp.at[sl])
        pltpu.sync_copy(ei_hbm.at[0, pl.ds(wid * per_w, per_w)], src_v)
        pltpu.sync_copy(ei_hbm.at[1, pl.ds(wid * per_w, per_w)], dst_v)
        plsc.subcore_barrier()

        for k in range(nbuf - 1):
            pltpu.async_copy(g_sp.at[src_v.at[k]], rows_v.at[k], gsem)

        def step(j, b):
            pltpu.make_async_copy(
                g_sp.at[src_v.at[j]], rows_v.at[b], gsem).wait()
            pltpu.async_copy(
                rows_v.at[b], agg_sp.at[dst_v.at[j]], ssem, add=True)

            @pl.when(j >= 1)
            def _():
                pb = (b + nbuf - 1) % nbuf
                pltpu.make_async_copy(
                    rows_v.at[pb], agg_sp.at[dst_v.at[j - 1]], ssem).wait()

            @pl.when(j + nbuf - 1 < per_w)
            def _():
                nb = (b + nbuf - 1) % nbuf
                pltpu.async_copy(
                    g_sp.at[src_v.at[j + nbuf - 1]], rows_v.at[nb], gsem)

        def body(jj, carry):
            j = jj * nbuf
            for k in range(nbuf):
                step(j + k, k)
            return carry

        lax.fori_loop(0, per_w // nbuf, body, 0)
        pltpu.make_async_copy(
            rows_v.at[(per_w - 1) % nbuf],
            agg_sp.at[dst_v.at[per_w - 1]], ssem).wait()
        plsc.subcore_barrier()
        pltpu.sync_copy(agg_sp.at[sl], out_hbm.at[cid, sl])

    return agg_kernel


def _tc1_body(dp_ref, xg_ref, w1k_ref, dinv_ref, g1_ref):
    deg = dp_ref[0] + dp_ref[1] + 1.0
    dinv = lax.rsqrt(deg)
    dinv_ref[...] = dinv
    h = jnp.dot(xg_ref[...], w1k_ref[...], preferred_element_type=jnp.float32)
    g1_ref[...] = h * dinv


def _tc2_body(n_valid, agg_ref, g1_ref, dinv_ref, b1_ref, w2k_ref, g2_ref):
    dinv = dinv_ref[...]
    s = agg_ref[0] + agg_ref[1] + g1_ref[...]
    z = jnp.maximum(dinv * s + b1_ref[...], 0.0)
    grp = lax.broadcasted_iota(jnp.int32, z.shape, 0)
    lane = lax.broadcasted_iota(jnp.int32, z.shape, 1)
    node = grp * PACK + lax.shift_right_logical(lane, 4)
    z = jnp.where(node < n_valid, z, 0.0)
    g2_ref[...] = jnp.dot(z, w2k_ref[...],
                          preferred_element_type=jnp.float32) * dinv


def _tc3_body(agg_ref, g2_ref, dinv_ref, b2_ref, onesk_ref, out_ref):
    y = dinv_ref[...] * (agg_ref[0] + agg_ref[1] + g2_ref[...]) + b2_ref[...]
    m = jnp.max(y, axis=1, keepdims=True)
    e = jnp.exp(y - m)
    s = jnp.dot(e, onesk_ref[...], preferred_element_type=jnp.float32)
    out_ref[...] = y - m - jnp.log(s)


def kernel(x, edge_index, W1, b1, W2, b2):
    n, d = x.shape
    e = edge_index.shape[1]
    h = W1.shape[1]
    c = W2.shape[1]
    assert h == F and c == F and n % PACK == 0

    n_pad = ((n + 511) // 512 + (1 if n % 512 == 0 else 0)) * 512
    g_rows = n_pad // PACK
    e_pad = ((e + NW * CHUNK * 8 - 1) // (NW * CHUNK * 8)) * (NW * CHUNK * 8)
    chunks_total = e_pad // CHUNK

    n_spread = min(128, n_pad - n)
    pad_idx = (n + (jnp.arange(e_pad, dtype=jnp.int32) & (n_spread - 1)))
    ei = jnp.broadcast_to(pad_idx, (2, e_pad)).at[:, :e].set(edge_index)
    ei = ei.reshape(2, chunks_total, CHUNK)
    x_pad = jnp.zeros((n_pad, d), x.dtype).at[:n].set(x)
    x_g = x_pad.reshape(g_rows, PACK * d)
    eye = jnp.eye(PACK, dtype=jnp.float32)
    w1k = jnp.kron(eye, W1)
    w2k = jnp.kron(eye, W2)
    onesk = jnp.kron(eye, jnp.ones((F, F), jnp.float32))
    b1p = jnp.tile(b1, PACK).reshape(1, 128)
    b2p = jnp.tile(b2, PACK).reshape(1, 128)
    zeros_rows = jnp.zeros((n_pad, F), jnp.float32)
    ones_rows = jnp.ones((CHUNK, F), jnp.float32)

    deg_parts = _make_deg_kernel(n_pad, chunks_total)(ei, zeros_rows, ones_rows)
    deg_p = deg_parts.reshape(NC, g_rows, 128)

    dinv, g1p = pl.pallas_call(
        _tc1_body,
        out_shape=(jax.ShapeDtypeStruct((g_rows, 128), jnp.float32),
                   jax.ShapeDtypeStruct((g_rows, 128), jnp.float32)),
    )(deg_p, x_g, w1k)

    agg_fn = _make_agg_kernel(n_pad, chunks_total)

    agg1 = agg_fn(g1p.reshape(n_pad, F), ei, zeros_rows)

    g2p = pl.pallas_call(
        functools.partial(_tc2_body, n),
        out_shape=jax.ShapeDtypeStruct((g_rows, 128), jnp.float32),
    )(agg1.reshape(NC, g_rows, 128), g1p, dinv, b1p, w2k)

    agg2 = agg_fn(g2p.reshape(n_pad, F), ei, zeros_rows)

    outp = pl.pallas_call(
        _tc3_body,
        out_shape=jax.ShapeDtypeStruct((g_rows, 128), jnp.float32),
    )(agg2.reshape(NC, g_rows, 128), g2p, dinv, b2p, onesk)

    return outp.reshape(n_pad, F)[:n]
